# baseline (speedup 1.0000x reference)
import functools
import jax
import jax.numpy as jnp
from jax import lax
from jax.experimental import pallas as pl
from jax.experimental.pallas import tpu as pltpu, tpu_sc as plsc

NW = 32
RX = 8
NBUF = 3
NWBUF = 3


def _make_sc(B, S, D):
    s_per_w = S // NW
    n_chunks = s_per_w // RX
    vecs = D // 16
    mesh = plsc.VectorSubcoreMesh(core_axis_name="c", subcore_axis_name="s")

    @functools.partial(
        pl.kernel,
        mesh=mesh,
        out_type=jax.ShapeDtypeStruct((B, S, D), jnp.float32),
        scratch_types=(
            [pltpu.VMEM((RX, D), jnp.float32) for _ in range(NWBUF)]
            + [pltpu.VMEM((RX, D), jnp.float32) for _ in range(B * NBUF)]
            + [pltpu.SemaphoreType.DMA
               for _ in range(NWBUF + 2 * B * NBUF)]
        ),
    )
    def k(x_hbm, wpe_hbm, out_hbm, *scratch):
        wbufs = scratch[0:NWBUF]
        xbufs = scratch[NWBUF:NWBUF + B * NBUF]
        sems = scratch[NWBUF + B * NBUF:]
        w_sems = sems[0:NWBUF]
        in_sems = sems[NWBUF:NWBUF + B * NBUF]
        out_sems = sems[NWBUF + B * NBUF:]
        wid = lax.axis_index("s") * 2 + lax.axis_index("c")
        s0 = wid * s_per_w

        def rows(ci):
            return pl.ds(s0 + ci * RX, RX)

        def start_ins(ci):
            sl = ci % NBUF
            return [
                pltpu.async_copy(x_hbm.at[(b, rows(ci))],
                                 xbufs[b * NBUF + sl],
                                 in_sems[b * NBUF + sl])
                for b in range(B)
            ]

        def start_w(ci):
            return pltpu.async_copy(wpe_hbm.at[rows(ci)],
                                    wbufs[ci % NWBUF], w_sems[ci % NWBUF])

        w_descs = [None] * n_chunks
        in_descs = [None] * n_chunks
        out_descs = [None] * n_chunks

        w_descs[0] = start_w(0)
        in_descs[0] = start_ins(0)
        w_descs[1] = start_w(1)
        in_descs[1] = start_ins(1)
        for ci in range(n_chunks):
            sl = ci % NBUF
            if ci >= 2:
                for d in out_descs[ci - 2]:
                    d.wait()
            if ci + 2 < n_chunks:
                w_descs[ci + 2] = start_w(ci + 2)
            if ci + 1 < n_chunks and ci + 1 >= 2:
                in_descs[ci + 1] = start_ins(ci + 1)
            w_descs[ci].wait()
            for d in in_descs[ci]:
                d.wait()

            wb = wbufs[ci % NWBUF]
            bbufs = [xbufs[b * NBUF + sl] for b in range(B)]

            @plsc.parallel_loop(0, RX, 1)
            def _(r):
                @plsc.parallel_loop(0, vecs, 1, unroll=8)
                def _(c):
                    wv = wb[r, pl.ds(c * 16, 16)]
                    for b in range(B):
                        plsc.addupdate(bbufs[b].at[r, pl.ds(c * 16, 16)], wv)

            out_descs[ci] = [
                pltpu.async_copy(bbufs[b], out_hbm.at[(b, rows(ci))],
                                 out_sems[b * NBUF + sl])
                for b in range(B)
            ]
        for d in out_descs[n_chunks - 2]:
            d.wait()
        for d in out_descs[n_chunks - 1]:
            d.wait()

    return k


def kernel(x, wpe):
    B, S, D = x.shape
    return _make_sc(B, S, D)(x, wpe)


# speedup vs baseline: 1.0535x; 1.0535x over previous
import functools
import jax
import jax.numpy as jnp
from jax import lax
from jax.experimental import pallas as pl
from jax.experimental.pallas import tpu as pltpu, tpu_sc as plsc

NW = 32
RX = 8
NBUF = 3


def _make_sc(B, S, D):
    s_per_w = S // NW
    n_chunks = s_per_w // RX
    vecs = D // 16
    nbufs = (B + 1) * NBUF
    mesh = plsc.VectorSubcoreMesh(core_axis_name="c", subcore_axis_name="s")

    @functools.partial(
        pl.kernel,
        mesh=mesh,
        out_type=jax.ShapeDtypeStruct((B, S, D), jnp.float32),
        scratch_types=(
            [pltpu.VMEM((nbufs * RX, D), jnp.float32)]
            + [pltpu.SemaphoreType.DMA for _ in range(3 * NBUF)]
        ),
    )
    def k(x_hbm, wpe_hbm, out_hbm, buf, *sems):
        w_sems = sems[0:NBUF]
        in_sems = sems[NBUF:2 * NBUF]
        out_sems = sems[2 * NBUF:]
        wid = lax.axis_index("s") * 2 + lax.axis_index("c")
        s0 = wid * s_per_w

        def rows(ci):
            return pl.ds(s0 + ci * RX, RX)

        def wslot(sl):
            return pl.ds(sl * RX, RX)

        def xslot(b, sl):
            return pl.ds((NBUF + b * NBUF + sl) * RX, RX)

        def start_ins(ci):
            sl = ci % NBUF
            return [
                pltpu.async_copy(x_hbm.at[(b, rows(ci))],
                                 buf.at[xslot(b, sl)], in_sems[sl])
                for b in range(B)
            ]

        def start_w(ci):
            sl = ci % NBUF
            return pltpu.async_copy(wpe_hbm.at[rows(ci)],
                                    buf.at[wslot(sl)], w_sems[sl])

        w_descs = [None] * n_chunks
        in_descs = [None] * n_chunks
        out_descs = [None] * n_chunks

        w_descs[0] = start_w(0)
        in_descs[0] = start_ins(0)
        w_descs[1] = start_w(1)
        in_descs[1] = start_ins(1)
        for ci in range(n_chunks):
            sl = ci % NBUF
            if ci >= 2:
                for d in out_descs[ci - 2]:
                    d.wait()
            if ci + 2 < n_chunks:
                w_descs[ci + 2] = start_w(ci + 2)
            if ci + 1 < n_chunks and ci + 1 >= 2:
                in_descs[ci + 1] = start_ins(ci + 1)
            w_descs[ci].wait()
            for d in in_descs[ci]:
                d.wait()

            @plsc.parallel_loop(0, RX, 1)
            def _(r):
                @plsc.parallel_loop(0, vecs, 1, unroll=4)
                def _(c):
                    wv = buf[sl * RX + r, pl.ds(c * 16, 16)]
                    for b in range(B):
                        plsc.addupdate(
                            buf.at[(NBUF + b * NBUF + sl) * RX + r,
                                   pl.ds(c * 16, 16)], wv)

            out_descs[ci] = [
                pltpu.async_copy(buf.at[xslot(b, sl)],
                                 out_hbm.at[(b, rows(ci))], out_sems[sl])
                for b in range(B)
            ]
        for d in out_descs[n_chunks - 2]:
            d.wait()
        for d in out_descs[n_chunks - 1]:
            d.wait()

    return k


def kernel(x, wpe):
    B, S, D = x.shape
    return _make_sc(B, S, D)(x, wpe)


# speedup vs baseline: 1.1014x; 1.0455x over previous
import functools
import jax
import jax.numpy as jnp
from jax import lax
from jax.experimental import pallas as pl
from jax.experimental.pallas import tpu as pltpu, tpu_sc as plsc

NW = 32
RX = 8
NBUF = 4
LEAD = 2


def _make_sc(B, S, D):
    s_per_w = S // NW
    n_chunks = s_per_w // RX
    vecs = D // 16
    nbufs = (B + 1) * NBUF
    mesh = plsc.VectorSubcoreMesh(core_axis_name="c", subcore_axis_name="s")

    @functools.partial(
        pl.kernel,
        mesh=mesh,
        out_type=jax.ShapeDtypeStruct((B, S, D), jnp.float32),
        scratch_types=(
            [pltpu.VMEM((nbufs * RX, D), jnp.float32)]
            + [pltpu.SemaphoreType.DMA for _ in range(2 * NBUF)]
        ),
    )
    def k(x_hbm, wpe_hbm, out_hbm, buf, *sems):
        io_sems = sems[0:NBUF]
        out_sems = sems[NBUF:]
        wid = lax.axis_index("s") * 2 + lax.axis_index("c")
        s0 = wid * s_per_w

        def rows(ci):
            return pl.ds(s0 + ci * RX, RX)

        def wslot(sl):
            return pl.ds(sl * RX, RX)

        def xslot(b, sl):
            return pl.ds((NBUF + b * NBUF + sl) * RX, RX)

        def w_copy(ci, sl):
            return pltpu.make_async_copy(wpe_hbm.at[rows(ci)],
                                         buf.at[wslot(sl)], io_sems[sl])

        def in_copy(b, ci, sl):
            return pltpu.make_async_copy(x_hbm.at[(b, rows(ci))],
                                         buf.at[xslot(b, sl)], io_sems[sl])

        def out_copy(b, ci, sl):
            return pltpu.make_async_copy(buf.at[xslot(b, sl)],
                                         out_hbm.at[(b, rows(ci))],
                                         out_sems[sl])

        def start_chunk(ci, sl):
            w_copy(ci, sl).start()
            for b in range(B):
                in_copy(b, ci, sl).start()

        for ci in range(LEAD):
            start_chunk(ci, ci % NBUF)

        @pl.loop(0, n_chunks, step=NBUF)
        def _(g):
            for k in range(NBUF):
                ci = g + k
                sl = k
                sl2 = (k + LEAD) % NBUF

                @pl.when(ci >= LEAD)
                def _():
                    for b in range(B):
                        out_copy(b, ci - LEAD, sl2).wait()

                @pl.when(ci + LEAD < n_chunks)
                def _():
                    start_chunk(ci + LEAD, sl2)

                w_copy(ci, sl).wait()
                for b in range(B):
                    in_copy(b, ci, sl).wait()

                @plsc.parallel_loop(0, RX, 1)
                def _(r):
                    @plsc.parallel_loop(0, vecs, 1, unroll=4)
                    def _(c):
                        wv = buf[sl * RX + r, pl.ds(c * 16, 16)]
                        for b in range(B):
                            plsc.addupdate(
                                buf.at[(NBUF + b * NBUF + sl) * RX + r,
                                       pl.ds(c * 16, 16)], wv)

                for b in range(B):
                    out_copy(b, ci, sl).start()

        for ci in range(n_chunks - LEAD, n_chunks):
            for b in range(B):
                out_copy(b, ci, ci % NBUF).wait()

    return k


def kernel(x, wpe):
    B, S, D = x.shape
    return _make_sc(B, S, D)(x, wpe)
